# baseline (speedup 1.0000x reference)
import jax
import jax.numpy as jnp
from jax import lax
from jax.experimental import pallas as pl
from jax.experimental.pallas import tpu as pltpu
from jax.experimental.pallas import tpu_sc as plsc

SEQ = 8192
D = 1024
L = 16
NC = 2
NS = 16
NW = NC * NS
RW = SEQ // NW
C = 32
NCH = RW // C


def _sc_body(x_hbm, pe_hbm, idx_hbm, out_hbm, idx_v, pe_buf, x_buf, sem):
    wid = lax.axis_index("s") * NC + lax.axis_index("c")
    base = wid * RW

    @pl.loop(0, NCH)
    def _chunk(ci):
        row0 = base + ci * C
        pltpu.sync_copy(idx_hbm.at[pl.ds(row0, C)], idx_v)
        gat = pltpu.async_copy(pe_hbm.at[idx_v], pe_buf, sem)
        pltpu.sync_copy(x_hbm.at[pl.ds(row0, C), :], x_buf)
        gat.wait()

        @pl.loop(0, C)
        def _row(r):
            @pl.loop(0, D // L, unroll=8)
            def _add(j):
                sl = pl.ds(j * L, L)
                x_buf[r, sl] = x_buf[r, sl] + pe_buf[r, sl]

        pltpu.sync_copy(x_buf, out_hbm.at[pl.ds(row0, C), :])


def kernel(x, pe, frame_indices):
    mesh = plsc.VectorSubcoreMesh(core_axis_name="c", subcore_axis_name="s")
    k = pl.kernel(
        _sc_body,
        out_type=jax.ShapeDtypeStruct((SEQ, D), jnp.float32),
        mesh=mesh,
        scratch_types=[
            pltpu.VMEM((C,), jnp.int32),
            pltpu.VMEM((C, D), jnp.float32),
            pltpu.VMEM((C, D), jnp.float32),
            pltpu.SemaphoreType.DMA,
        ],
    )
    return k(x, pe, frame_indices)


# speedup vs baseline: 1.2626x; 1.2626x over previous
import jax
import jax.numpy as jnp
from jax import lax
from jax.experimental import pallas as pl
from jax.experimental.pallas import tpu as pltpu
from jax.experimental.pallas import tpu_sc as plsc

SEQ = 8192
D = 1024
L = 16
NC = 2
NS = 16
NW = NC * NS
RW = SEQ // NW
C = 16
NCH = RW // C
NB = 2


def _sc_body(x_hbm, pe_hbm, idx_hbm, out_hbm,
             idx_all, pe_bufs, x_bufs, o_bufs, gsems, xsems, osems):
    wid = lax.axis_index("s") * NC + lax.axis_index("c")
    base = wid * RW

    pltpu.sync_copy(idx_hbm.at[pl.ds(base, RW)], idx_all)

    def start_loads(ci, b):
        row0 = base + ci * C
        pltpu.async_copy(pe_hbm.at[idx_all.at[pl.ds(ci * C, C)]],
                         pe_bufs[b], gsems[b])
        pltpu.async_copy(x_hbm.at[pl.ds(row0, C), :], x_bufs[b], xsems[b])

    for b in range(NB):
        start_loads(b, b)

    @pl.loop(0, NCH, step=NB)
    def _outer(ci0):
        for b in range(NB):
            ci = ci0 + b
            pltpu.make_async_copy(pe_hbm.at[idx_all.at[pl.ds(0, C)]],
                                  pe_bufs[b], gsems[b]).wait()
            pltpu.make_async_copy(x_hbm.at[pl.ds(0, C), :],
                                  x_bufs[b], xsems[b]).wait()

            @pl.when(ci >= NB)
            def _():
                pltpu.make_async_copy(o_bufs[b], out_hbm.at[pl.ds(0, C), :],
                                      osems[b]).wait()

            @pl.loop(0, C)
            def _row(r):
                @pl.loop(0, D // L, unroll=8)
                def _add(j):
                    sl = pl.ds(j * L, L)
                    o_bufs[b][r, sl] = x_bufs[b][r, sl] + pe_bufs[b][r, sl]

            @pl.when(ci + NB < NCH)
            def _():
                start_loads(ci + NB, b)

            row0 = base + ci * C
            pltpu.async_copy(o_bufs[b], out_hbm.at[pl.ds(row0, C), :],
                             osems[b])

    for b in range(NB):
        pltpu.make_async_copy(o_bufs[b], out_hbm.at[pl.ds(0, C), :],
                              osems[b]).wait()


def kernel(x, pe, frame_indices):
    mesh = plsc.VectorSubcoreMesh(core_axis_name="c", subcore_axis_name="s")
    k = pl.kernel(
        _sc_body,
        out_type=jax.ShapeDtypeStruct((SEQ, D), jnp.float32),
        mesh=mesh,
        scratch_types=[
            pltpu.VMEM((RW,), jnp.int32),
            [pltpu.VMEM((C, D), jnp.float32) for _ in range(NB)],
            [pltpu.VMEM((C, D), jnp.float32) for _ in range(NB)],
            [pltpu.VMEM((C, D), jnp.float32) for _ in range(NB)],
            [pltpu.SemaphoreType.DMA for _ in range(NB)],
            [pltpu.SemaphoreType.DMA for _ in range(NB)],
            [pltpu.SemaphoreType.DMA for _ in range(NB)],
        ],
    )
    return k(x, pe, frame_indices)


# speedup vs baseline: 2.4909x; 1.9729x over previous
import jax
import jax.numpy as jnp
from jax import lax
from jax.experimental import pallas as pl
from jax.experimental.pallas import tpu as pltpu
from jax.experimental.pallas import tpu_sc as plsc

SEQ = 8192
D = 1024
L = 16
NC = 2
NS = 16
NW = NC * NS
RW = SEQ // NW
C = 8
NCH = RW // C
NB = 4


def _sc_body(x_hbm, pe_hbm, idx_hbm, out_hbm, idx_all, x_bufs, pe_bufs,
             xsems, gsems, osems):
    wid = lax.axis_index("s") * NC + lax.axis_index("c")
    base = wid * RW

    pltpu.sync_copy(idx_hbm.at[pl.ds(base, RW)], idx_all)

    def lstart(c, b):
        pltpu.async_copy(x_hbm.at[pl.ds(base + c * C, C), :], x_bufs[b],
                         xsems[b])
        pltpu.async_copy(pe_hbm.at[idx_all.at[pl.ds(c * C, C)]], pe_bufs[b],
                         gsems[b])

    def lwait(b):
        pltpu.make_async_copy(x_hbm.at[pl.ds(0, C), :], x_bufs[b],
                              xsems[b]).wait()
        pltpu.make_async_copy(pe_hbm.at[idx_all.at[pl.ds(0, C)]], pe_bufs[b],
                              gsems[b]).wait()

    def ostart(c, b):
        pltpu.async_copy(x_bufs[b], out_hbm.at[pl.ds(base + c * C, C), :],
                         osems[b])

    def owait(b):
        pltpu.make_async_copy(x_bufs[b], out_hbm.at[pl.ds(0, C), :],
                              osems[b]).wait()

    for c in range(NB - 1):
        lstart(c, c)

    @pl.loop(0, NCH, step=NB)
    def _outer(ci0):
        for b in range(NB):
            ci = ci0 + b
            pb = (b - 1) % NB

            if b == 0:
                @pl.when(ci0 >= 1)
                def _():
                    owait(pb)
            else:
                owait(pb)

            @pl.when(ci + NB - 1 < NCH)
            def _():
                lstart(ci + NB - 1, pb)

            lwait(b)

            @pl.loop(0, C)
            def _row(r):
                @pl.loop(0, D // L, unroll=8)
                def _add(j):
                    sl = pl.ds(j * L, L)
                    plsc.addupdate(x_bufs[b].at[r, sl], pe_bufs[b][r, sl])

            ostart(ci, b)

    owait((NCH - 1) % NB)


def kernel(x, pe, frame_indices):
    mesh = plsc.VectorSubcoreMesh(core_axis_name="c", subcore_axis_name="s")
    k = pl.kernel(
        _sc_body,
        out_type=jax.ShapeDtypeStruct((SEQ, D), jnp.float32),
        mesh=mesh,
        scratch_types=[
            pltpu.VMEM((RW,), jnp.int32),
            [pltpu.VMEM((C, D), jnp.float32) for _ in range(NB)],
            [pltpu.VMEM((C, D), jnp.float32) for _ in range(NB)],
            [pltpu.SemaphoreType.DMA for _ in range(NB)],
            [pltpu.SemaphoreType.DMA for _ in range(NB)],
            [pltpu.SemaphoreType.DMA for _ in range(NB)],
        ],
    )
    return k(x, pe, frame_indices)
